# Initial kernel scaffold; baseline (speedup 1.0000x reference)
import jax
import jax.numpy as jnp
from jax.experimental import pallas as pl


def kernel(hidden_states, position_ids, Wq, Wk, Wv, Wo, W1, W2, eps):
    raise NotImplementedError("write your pallas kernel here")


# speedup vs baseline: 4.3060x; 4.3060x over previous
import jax
import jax.numpy as jnp
from jax.experimental import pallas as pl
from jax.experimental.pallas import tpu as pltpu

S_, D_ = 2048, 768
H_, HD_ = 12, 64
MLP_HID_ = 128
AGG_IN_ = 4 * HD_
THETA = 10000.0
NEG_INF = -3.0e38

BQ = 256
NQ = S_ // BQ
BS_OUT = 512


def _qkv_kernel(x_ref, pos_ref, wq_ref, wk_ref, wv_ref, q_ref, k_ref, v_ref):
    x = x_ref[...]
    pos = pos_ref[...].astype(jnp.float32)
    i = jax.lax.broadcasted_iota(jnp.int32, (1, HD_ // 2), 1).astype(jnp.float32)
    inv_freq = jnp.exp(i * (-2.0 * jnp.log(THETA) / HD_))
    ang = pos * inv_freq
    cos_a = jnp.cos(ang)
    sin_a = jnp.sin(ang)

    def proj(w_ref):
        return jax.lax.dot_general(x, w_ref[0], (((1,), (0,)), ((), ())),
                                   preferred_element_type=jnp.float32)

    def rope(t):
        t1 = t[:, :HD_ // 2]
        t2 = t[:, HD_ // 2:]
        return jnp.concatenate(
            [t1 * cos_a - t2 * sin_a, t2 * cos_a + t1 * sin_a], axis=1)

    q_ref[0] = rope(proj(wq_ref))
    k_ref[0] = rope(proj(wk_ref))
    v_ref[0] = proj(wv_ref)


def _attn_kernel(q_ref, k_ref, v_ref, w1_ref, w2_ref, o_ref, dinv_ref, mx_ref):
    qi = pl.program_id(1)

    @pl.when(qi == 0)
    def _init():
        dinv_ref[...] = jnp.zeros_like(dinv_ref)
        mx_ref[...] = jnp.full_like(mx_ref, NEG_INF)

    qb = q_ref[0]
    kf = k_ref[0]
    vf = v_ref[0]

    s = jax.lax.dot_general(qb, kf, (((1,), (1,)), ((), ())),
                            preferred_element_type=jnp.float32)
    s = s * (1.0 / 8.0)
    r = jax.lax.broadcasted_iota(jnp.int32, (BQ, S_), 0) + qi * BQ
    c = jax.lax.broadcasted_iota(jnp.int32, (BQ, S_), 1)
    a = jnp.where(c <= r, jnp.maximum(s, 0.0), 0.0)

    deg = jnp.sum(a, axis=1, keepdims=True)
    dinv = jnp.where(deg > 0.0, jax.lax.rsqrt(deg), 0.0)
    dinv_ref[pl.ds(qi * BQ, BQ), :] = dinv
    dcol = dinv_ref[...]

    v3 = jnp.concatenate([vf, vf * dcol, vf * vf * dcol], axis=1)
    pv = jax.lax.dot_general(a, v3, (((1,), (0,)), ((), ())),
                             preferred_element_type=jnp.float32)
    sum_agg = pv[:, :HD_]
    mean_agg = pv[:, HD_:2 * HD_] * dinv
    mean_sq = pv[:, 2 * HD_:] * dinv
    var_agg = jnp.maximum(mean_sq - mean_agg * mean_agg, 0.0)

    m = v_ref[0, pl.ds(qi * BQ, BQ), :]
    step = 1
    while step < BQ:
        pad = jnp.full((step, HD_), NEG_INF, jnp.float32)
        m = jnp.maximum(m, jnp.concatenate([pad, m[:-step]], axis=0))
        step *= 2
    m = jnp.maximum(m, mx_ref[...])
    mx_ref[...] = m[BQ - 1:BQ, :]

    agg = jnp.concatenate([sum_agg, mean_agg, m, var_agg], axis=1)
    h1 = jax.lax.dot_general(agg, w1_ref[0], (((1,), (0,)), ((), ())),
                             preferred_element_type=jnp.float32)
    h1 = h1 * jax.nn.sigmoid(h1)
    o_ref[0] = jax.lax.dot_general(h1, w2_ref[0], (((1,), (0,)), ((), ())),
                                   preferred_element_type=jnp.float32)


def _out_kernel(x_ref, wo_ref, hid_ref, eps_ref, o_ref):
    h = pl.program_id(1)
    e = eps_ref[0]

    @pl.when(h == 0)
    def _init():
        o_ref[...] = e * hid_ref[...]

    y = jax.lax.dot_general(x_ref[0], wo_ref[0], (((1,), (0,)), ((), ())),
                            preferred_element_type=jnp.float32)
    o_ref[...] += (1.0 - e) * y


def kernel(hidden_states, position_ids, Wq, Wk, Wv, Wo, W1, W2, eps):
    x = hidden_states.reshape(S_, D_)
    pos = position_ids.reshape(S_, 1)
    wq3 = Wq.reshape(D_, H_, HD_).transpose(1, 0, 2)
    wk3 = Wk.reshape(D_, H_, HD_).transpose(1, 0, 2)
    wv3 = Wv.reshape(D_, H_, HD_).transpose(1, 0, 2)
    wo3 = Wo.reshape(H_, HD_, D_)

    q, k, v = pl.pallas_call(
        _qkv_kernel,
        grid=(H_,),
        in_specs=[
            pl.BlockSpec((S_, D_), lambda h: (0, 0)),
            pl.BlockSpec((S_, 1), lambda h: (0, 0)),
            pl.BlockSpec((1, D_, HD_), lambda h: (h, 0, 0)),
            pl.BlockSpec((1, D_, HD_), lambda h: (h, 0, 0)),
            pl.BlockSpec((1, D_, HD_), lambda h: (h, 0, 0)),
        ],
        out_specs=[
            pl.BlockSpec((1, S_, HD_), lambda h: (h, 0, 0)),
            pl.BlockSpec((1, S_, HD_), lambda h: (h, 0, 0)),
            pl.BlockSpec((1, S_, HD_), lambda h: (h, 0, 0)),
        ],
        out_shape=[jax.ShapeDtypeStruct((H_, S_, HD_), jnp.float32)] * 3,
    )(x, pos, wq3, wk3, wv3)

    attn = pl.pallas_call(
        _attn_kernel,
        grid=(H_, NQ),
        in_specs=[
            pl.BlockSpec((1, BQ, HD_), lambda h, qi: (h, qi, 0)),
            pl.BlockSpec((1, S_, HD_), lambda h, qi: (h, 0, 0)),
            pl.BlockSpec((1, S_, HD_), lambda h, qi: (h, 0, 0)),
            pl.BlockSpec((1, AGG_IN_, MLP_HID_), lambda h, qi: (h, 0, 0)),
            pl.BlockSpec((1, MLP_HID_, HD_), lambda h, qi: (h, 0, 0)),
        ],
        out_specs=pl.BlockSpec((1, BQ, HD_), lambda h, qi: (h, qi, 0)),
        out_shape=jax.ShapeDtypeStruct((H_, S_, HD_), jnp.float32),
        scratch_shapes=[
            pltpu.VMEM((S_, 1), jnp.float32),
            pltpu.VMEM((1, HD_), jnp.float32),
        ],
    )(q, k, v, W1, W2)

    out = pl.pallas_call(
        _out_kernel,
        grid=(S_ // BS_OUT, H_),
        in_specs=[
            pl.BlockSpec((1, BS_OUT, HD_), lambda si, h: (h, si, 0)),
            pl.BlockSpec((1, HD_, D_), lambda si, h: (h, 0, 0)),
            pl.BlockSpec((BS_OUT, D_), lambda si, h: (si, 0)),
            pl.BlockSpec(memory_space=pltpu.SMEM),
        ],
        out_specs=pl.BlockSpec((BS_OUT, D_), lambda si, h: (si, 0)),
        out_shape=jax.ShapeDtypeStruct((S_, D_), jnp.float32),
    )(attn, wo3, x, jnp.reshape(eps, (1,)))

    return out.reshape(1, S_, D_)


# speedup vs baseline: 4.7930x; 1.1131x over previous
import jax
import jax.numpy as jnp
from jax.experimental import pallas as pl
from jax.experimental.pallas import tpu as pltpu

S_, D_ = 2048, 768
H_, HD_ = 12, 64
MLP_HID_ = 128
AGG_IN_ = 4 * HD_
THETA = 10000.0
NEG_INF = -3.0e38

BQ = 256
NQ = S_ // BQ
BS_OUT = 512


def _qkv_kernel(x_ref, pos_ref, wq_ref, wk_ref, wv_ref, q_ref, k_ref, v_ref,
                cos_ref, sin_ref):
    h = pl.program_id(0)

    @pl.when(h == 0)
    def _trig():
        pos = pos_ref[...].astype(jnp.float32)
        i = jax.lax.broadcasted_iota(
            jnp.int32, (1, HD_ // 2), 1).astype(jnp.float32)
        inv_freq = jnp.exp(i * (-2.0 * jnp.log(THETA) / HD_))
        ang = pos * inv_freq
        cos_ref[...] = jnp.cos(ang)
        sin_ref[...] = jnp.sin(ang)

    x = x_ref[...]
    cos_a = cos_ref[...]
    sin_a = sin_ref[...]

    def proj(w_ref):
        return jax.lax.dot_general(x, w_ref[0], (((1,), (0,)), ((), ())),
                                   preferred_element_type=jnp.float32)

    def rope(t):
        t1 = t[:, :HD_ // 2]
        t2 = t[:, HD_ // 2:]
        return jnp.concatenate(
            [t1 * cos_a - t2 * sin_a, t2 * cos_a + t1 * sin_a], axis=1)

    q_ref[0] = rope(proj(wq_ref)) * 0.125
    k_ref[0] = rope(proj(wk_ref))
    v_ref[0] = proj(wv_ref)


def _attn_kernel(q_ref, k_ref, v_ref, w1_ref, w2_ref, o_ref,
                 v3_ref, mx_ref, pv_ref):
    h = pl.program_id(0)
    qi = pl.program_id(1)

    @pl.when((h == 0) & (qi == 0))
    def _init_v3():
        v3_ref[...] = jnp.zeros_like(v3_ref)

    @pl.when(qi == 0)
    def _init_mx():
        mx_ref[...] = jnp.full_like(mx_ref, NEG_INF)

    qb = q_ref[0]
    pv_ref[...] = jnp.zeros_like(pv_ref)

    def body(t, deg):
        kc = k_ref[0, pl.ds(t * BQ, BQ), :]
        a_c = jnp.maximum(
            jax.lax.dot_general(qb, kc, (((1,), (1,)), ((), ())),
                                preferred_element_type=jnp.float32), 0.0)
        pv_ref[...] += jax.lax.dot_general(
            a_c, v3_ref[pl.ds(t * BQ, BQ), :], (((1,), (0,)), ((), ())),
            preferred_element_type=jnp.float32)
        return deg + jnp.sum(a_c, axis=1, keepdims=True)

    deg = jax.lax.fori_loop(0, qi, body, jnp.zeros((BQ, 1), jnp.float32))

    kd = k_ref[0, pl.ds(qi * BQ, BQ), :]
    s_d = jax.lax.dot_general(qb, kd, (((1,), (1,)), ((), ())),
                              preferred_element_type=jnp.float32)
    tri = (jax.lax.broadcasted_iota(jnp.int32, (BQ, BQ), 0)
           >= jax.lax.broadcasted_iota(jnp.int32, (BQ, BQ), 1))
    a_d = jnp.where(tri, jnp.maximum(s_d, 0.0), 0.0)
    deg = deg + jnp.sum(a_d, axis=1, keepdims=True)
    dinv = jnp.where(deg > 0.0, jax.lax.rsqrt(deg), 0.0)

    vb = v_ref[0, pl.ds(qi * BQ, BQ), :]
    v3_ref[pl.ds(qi * BQ, BQ), :] = jnp.concatenate(
        [vb, vb * dinv, vb * vb * dinv], axis=1)
    pv = pv_ref[...] + jax.lax.dot_general(
        a_d, v3_ref[pl.ds(qi * BQ, BQ), :], (((1,), (0,)), ((), ())),
        preferred_element_type=jnp.float32)
    sum_agg = pv[:, :HD_]
    mean_agg = pv[:, HD_:2 * HD_] * dinv
    mean_sq = pv[:, 2 * HD_:] * dinv
    var_agg = jnp.maximum(mean_sq - mean_agg * mean_agg, 0.0)

    m = v_ref[0, pl.ds(qi * BQ, BQ), :]
    step = 1
    while step < BQ:
        pad = jnp.full((step, HD_), NEG_INF, jnp.float32)
        m = jnp.maximum(m, jnp.concatenate([pad, m[:-step]], axis=0))
        step *= 2
    m = jnp.maximum(m, mx_ref[...])
    mx_ref[...] = m[BQ - 1:BQ, :]

    agg = jnp.concatenate([sum_agg, mean_agg, m, var_agg], axis=1)
    h1 = jax.lax.dot_general(agg, w1_ref[0], (((1,), (0,)), ((), ())),
                             preferred_element_type=jnp.float32)
    h1 = h1 * jax.nn.sigmoid(h1)
    o_ref[0] = jax.lax.dot_general(h1, w2_ref[0], (((1,), (0,)), ((), ())),
                                   preferred_element_type=jnp.float32)


def _out_kernel(x_ref, wo_ref, hid_ref, eps_ref, o_ref):
    h = pl.program_id(1)
    e = eps_ref[0]

    @pl.when(h == 0)
    def _init():
        o_ref[...] = e * hid_ref[...]

    y = jax.lax.dot_general(x_ref[0], wo_ref[0], (((1,), (0,)), ((), ())),
                            preferred_element_type=jnp.float32)
    o_ref[...] += (1.0 - e) * y


def kernel(hidden_states, position_ids, Wq, Wk, Wv, Wo, W1, W2, eps):
    x = hidden_states.reshape(S_, D_)
    pos = position_ids.reshape(S_, 1)
    wq3 = Wq.reshape(D_, H_, HD_).transpose(1, 0, 2)
    wk3 = Wk.reshape(D_, H_, HD_).transpose(1, 0, 2)
    wv3 = Wv.reshape(D_, H_, HD_).transpose(1, 0, 2)
    wo3 = Wo.reshape(H_, HD_, D_)

    q, k, v = pl.pallas_call(
        _qkv_kernel,
        grid=(H_,),
        in_specs=[
            pl.BlockSpec((S_, D_), lambda h: (0, 0)),
            pl.BlockSpec((S_, 1), lambda h: (0, 0)),
            pl.BlockSpec((1, D_, HD_), lambda h: (h, 0, 0)),
            pl.BlockSpec((1, D_, HD_), lambda h: (h, 0, 0)),
            pl.BlockSpec((1, D_, HD_), lambda h: (h, 0, 0)),
        ],
        out_specs=[
            pl.BlockSpec((1, S_, HD_), lambda h: (h, 0, 0)),
            pl.BlockSpec((1, S_, HD_), lambda h: (h, 0, 0)),
            pl.BlockSpec((1, S_, HD_), lambda h: (h, 0, 0)),
        ],
        out_shape=[jax.ShapeDtypeStruct((H_, S_, HD_), jnp.float32)] * 3,
        scratch_shapes=[
            pltpu.VMEM((S_, HD_ // 2), jnp.float32),
            pltpu.VMEM((S_, HD_ // 2), jnp.float32),
        ],
    )(x, pos, wq3, wk3, wv3)

    attn = pl.pallas_call(
        _attn_kernel,
        grid=(H_, NQ),
        in_specs=[
            pl.BlockSpec((1, BQ, HD_), lambda h, qi: (h, qi, 0)),
            pl.BlockSpec((1, S_, HD_), lambda h, qi: (h, 0, 0)),
            pl.BlockSpec((1, S_, HD_), lambda h, qi: (h, 0, 0)),
            pl.BlockSpec((1, AGG_IN_, MLP_HID_), lambda h, qi: (h, 0, 0)),
            pl.BlockSpec((1, MLP_HID_, HD_), lambda h, qi: (h, 0, 0)),
        ],
        out_specs=pl.BlockSpec((1, BQ, HD_), lambda h, qi: (h, qi, 0)),
        out_shape=jax.ShapeDtypeStruct((H_, S_, HD_), jnp.float32),
        scratch_shapes=[
            pltpu.VMEM((S_, 3 * HD_), jnp.float32),
            pltpu.VMEM((1, HD_), jnp.float32),
            pltpu.VMEM((BQ, 3 * HD_), jnp.float32),
        ],
    )(q, k, v, W1, W2)

    out = pl.pallas_call(
        _out_kernel,
        grid=(S_ // BS_OUT, H_),
        in_specs=[
            pl.BlockSpec((1, BS_OUT, HD_), lambda si, h: (h, si, 0)),
            pl.BlockSpec((1, HD_, D_), lambda si, h: (h, 0, 0)),
            pl.BlockSpec((BS_OUT, D_), lambda si, h: (si, 0)),
            pl.BlockSpec(memory_space=pltpu.SMEM),
        ],
        out_specs=pl.BlockSpec((BS_OUT, D_), lambda si, h: (si, 0)),
        out_shape=jax.ShapeDtypeStruct((S_, D_), jnp.float32),
    )(attn, wo3, x, jnp.reshape(eps, (1,)))

    return out.reshape(1, S_, D_)


# speedup vs baseline: 8.8851x; 1.8538x over previous
import jax
import jax.numpy as jnp
from jax.experimental import pallas as pl
from jax.experimental.pallas import tpu as pltpu

S_, D_ = 2048, 768
H_, HD_ = 12, 64
MLP_HID_ = 128
AGG_IN_ = 4 * HD_
THETA = 10000.0
NEG_INF = -3.0e38

BQ = 256
NQ = S_ // BQ
VW = 4 * HD_


def _mega_kernel(x_ref, pos_ref, wq_ref, wk_ref, wv_ref, w1_ref, w2_ref,
                 wo_ref, eps_ref, o_ref, v3_ref, cos_ref, sin_ref):
    h = pl.program_id(0)
    e = eps_ref[0]

    @pl.when(h == 0)
    def _trig():
        pos = pos_ref[...].astype(jnp.float32)
        i = jax.lax.broadcasted_iota(
            jnp.int32, (1, HD_ // 2), 1).astype(jnp.float32)
        inv_freq = jnp.exp(i * (-2.0 * jnp.log(THETA) / HD_))
        ang = pos * inv_freq
        cos_ref[...] = jnp.cos(ang)
        sin_ref[...] = jnp.sin(ang)

    x = x_ref[...]
    cos_a = cos_ref[...]
    sin_a = sin_ref[...]

    def proj(w_ref):
        return jax.lax.dot_general(x, w_ref[0], (((1,), (0,)), ((), ())),
                                   preferred_element_type=jnp.float32)

    def rope(t):
        t1 = t[:, :HD_ // 2]
        t2 = t[:, HD_ // 2:]
        return jnp.concatenate(
            [t1 * cos_a - t2 * sin_a, t2 * cos_a + t1 * sin_a], axis=1)

    qh = rope(proj(wq_ref)) * 0.125
    kh = rope(proj(wk_ref))
    vh = proj(wv_ref)

    tri = (jax.lax.broadcasted_iota(jnp.int32, (BQ, BQ), 0)
           >= jax.lax.broadcasted_iota(jnp.int32, (BQ, BQ), 1))
    carry = jnp.full((1, HD_), NEG_INF, jnp.float32)
    aggs = []
    for qi in range(NQ):
        lo = qi * BQ
        qb = qh[lo:lo + BQ, :]
        pv = jnp.zeros((BQ, VW), jnp.float32)
        for t in range(qi):
            kc = kh[t * BQ:(t + 1) * BQ, :]
            a_t = jnp.maximum(
                jax.lax.dot_general(qb, kc, (((1,), (1,)), ((), ())),
                                    preferred_element_type=jnp.float32), 0.0)
            pv = pv + jax.lax.dot_general(
                a_t, v3_ref[t * BQ:(t + 1) * BQ, :], (((1,), (0,)), ((), ())),
                preferred_element_type=jnp.float32)
        deg = pv[:, 3 * HD_:3 * HD_ + 1]
        kd = kh[lo:lo + BQ, :]
        s_d = jax.lax.dot_general(qb, kd, (((1,), (1,)), ((), ())),
                                  preferred_element_type=jnp.float32)
        a_d = jnp.where(tri, jnp.maximum(s_d, 0.0), 0.0)
        deg = deg + jnp.sum(a_d, axis=1, keepdims=True)
        dinv = jnp.where(deg > 0.0, jax.lax.rsqrt(deg), 0.0)

        vb = vh[lo:lo + BQ, :]
        v3b = jnp.concatenate(
            [vb, vb * dinv, vb * vb * dinv,
             jnp.ones((BQ, 1), jnp.float32),
             jnp.zeros((BQ, HD_ - 1), jnp.float32)], axis=1)
        v3_ref[lo:lo + BQ, :] = v3b
        pv = pv + jax.lax.dot_general(a_d, v3b, (((1,), (0,)), ((), ())),
                                      preferred_element_type=jnp.float32)

        sum_agg = pv[:, :HD_]
        mean_agg = pv[:, HD_:2 * HD_] * dinv
        mean_sq = pv[:, 2 * HD_:3 * HD_] * dinv
        var_agg = jnp.maximum(mean_sq - mean_agg * mean_agg, 0.0)

        m = vb
        step = 1
        while step < BQ:
            pad = jnp.full((step, HD_), NEG_INF, jnp.float32)
            m = jnp.maximum(m, jnp.concatenate([pad, m[:-step]], axis=0))
            step *= 2
        m = jnp.maximum(m, carry)
        carry = m[BQ - 1:BQ, :]

        aggs.append(jnp.concatenate([sum_agg, mean_agg, m, var_agg], axis=1))

    agg = jnp.concatenate(aggs, axis=0)
    h1 = jax.lax.dot_general(agg, w1_ref[0], (((1,), (0,)), ((), ())),
                             preferred_element_type=jnp.float32)
    h1 = h1 * jax.nn.sigmoid(h1)
    oh = jax.lax.dot_general(h1, w2_ref[0], (((1,), (0,)), ((), ())),
                             preferred_element_type=jnp.float32)
    y = jax.lax.dot_general(oh, wo_ref[0], (((1,), (0,)), ((), ())),
                            preferred_element_type=jnp.float32)

    @pl.when(h == 0)
    def _first():
        o_ref[...] = e * x + (1.0 - e) * y

    @pl.when(h > 0)
    def _rest():
        o_ref[...] += (1.0 - e) * y


def kernel(hidden_states, position_ids, Wq, Wk, Wv, Wo, W1, W2, eps):
    x = hidden_states.reshape(S_, D_)
    pos = position_ids.reshape(S_, 1)
    wq3 = Wq.reshape(D_, H_, HD_).transpose(1, 0, 2)
    wk3 = Wk.reshape(D_, H_, HD_).transpose(1, 0, 2)
    wv3 = Wv.reshape(D_, H_, HD_).transpose(1, 0, 2)
    wo3 = Wo.reshape(H_, HD_, D_)

    out = pl.pallas_call(
        _mega_kernel,
        grid=(H_,),
        in_specs=[
            pl.BlockSpec((S_, D_), lambda h: (0, 0)),
            pl.BlockSpec((S_, 1), lambda h: (0, 0)),
            pl.BlockSpec((1, D_, HD_), lambda h: (h, 0, 0)),
            pl.BlockSpec((1, D_, HD_), lambda h: (h, 0, 0)),
            pl.BlockSpec((1, D_, HD_), lambda h: (h, 0, 0)),
            pl.BlockSpec((1, AGG_IN_, MLP_HID_), lambda h: (h, 0, 0)),
            pl.BlockSpec((1, MLP_HID_, HD_), lambda h: (h, 0, 0)),
            pl.BlockSpec((1, HD_, D_), lambda h: (h, 0, 0)),
            pl.BlockSpec(memory_space=pltpu.SMEM),
        ],
        out_specs=pl.BlockSpec((S_, D_), lambda h: (0, 0)),
        out_shape=jax.ShapeDtypeStruct((S_, D_), jnp.float32),
        scratch_shapes=[
            pltpu.VMEM((S_, VW), jnp.float32),
            pltpu.VMEM((S_, HD_ // 2), jnp.float32),
            pltpu.VMEM((S_, HD_ // 2), jnp.float32),
        ],
    )(x, pos, wq3, wk3, wv3, W1, W2, wo3, jnp.reshape(eps, (1,)))

    return out.reshape(1, S_, D_)


# speedup vs baseline: 8.9994x; 1.0129x over previous
import jax
import jax.numpy as jnp
from jax.experimental import pallas as pl
from jax.experimental.pallas import tpu as pltpu

S_, D_ = 2048, 768
H_, HD_ = 12, 64
MLP_HID_ = 128
AGG_IN_ = 4 * HD_
THETA = 10000.0
NEG_INF = -3.0e38

BQ = 256
NQ = S_ // BQ
VW = 4 * HD_


def _dot(a, b, dims):
    return jax.lax.dot_general(a.astype(jnp.bfloat16), b.astype(jnp.bfloat16),
                               dims, preferred_element_type=jnp.float32)


def _mega_kernel(x_ref, xb_ref, pos_ref, wqk_ref, wv_ref, w1_ref, w2_ref,
                 wo_ref, eps_ref, o_ref, v3_ref, cos_ref, sin_ref):
    h = pl.program_id(0)
    e = eps_ref[0]

    @pl.when(h == 0)
    def _trig():
        pos = pos_ref[...].astype(jnp.float32)
        i = jax.lax.broadcasted_iota(
            jnp.int32, (1, HD_ // 2), 1).astype(jnp.float32)
        inv_freq = jnp.exp(i * (-2.0 * jnp.log(THETA) / HD_))
        ang = pos * inv_freq
        cos_ref[...] = jnp.cos(ang)
        sin_ref[...] = jnp.sin(ang)

    cos_a = cos_ref[...]
    sin_a = sin_ref[...]

    qk = jax.lax.dot_general(x_ref[...], wqk_ref[0], (((1,), (0,)), ((), ())),
                             preferred_element_type=jnp.float32)
    vh = jax.lax.dot_general(xb_ref[...], wv_ref[0], (((1,), (0,)), ((), ())),
                             preferred_element_type=jnp.float32)

    def rope(t):
        t1 = t[:, :HD_ // 2]
        t2 = t[:, HD_ // 2:]
        return jnp.concatenate(
            [t1 * cos_a - t2 * sin_a, t2 * cos_a + t1 * sin_a], axis=1)

    qh = rope(qk[:, :HD_]) * 0.125
    kh = rope(qk[:, HD_:])

    tri = (jax.lax.broadcasted_iota(jnp.int32, (BQ, BQ), 0)
           >= jax.lax.broadcasted_iota(jnp.int32, (BQ, BQ), 1))
    carry = jnp.full((1, HD_), NEG_INF, jnp.float32)
    aggs = []
    for qi in range(NQ):
        lo = qi * BQ
        qb = qh[lo:lo + BQ, :]
        pv = jnp.zeros((BQ, VW), jnp.float32)
        for t in range(qi):
            kc = kh[t * BQ:(t + 1) * BQ, :]
            a_t = jnp.maximum(
                jax.lax.dot_general(qb, kc, (((1,), (1,)), ((), ())),
                                    preferred_element_type=jnp.float32), 0.0)
            pv = pv + _dot(a_t, v3_ref[t * BQ:(t + 1) * BQ, :],
                           (((1,), (0,)), ((), ())))
        deg = pv[:, 3 * HD_:3 * HD_ + 1]
        kd = kh[lo:lo + BQ, :]
        s_d = jax.lax.dot_general(qb, kd, (((1,), (1,)), ((), ())),
                                  preferred_element_type=jnp.float32)
        a_d = jnp.where(tri, jnp.maximum(s_d, 0.0), 0.0)
        deg = deg + jnp.sum(a_d, axis=1, keepdims=True)
        dinv = jnp.where(deg > 0.0, jax.lax.rsqrt(deg), 0.0)

        vb = vh[lo:lo + BQ, :]
        v3b = jnp.concatenate(
            [vb, vb * dinv, vb * vb * dinv,
             jnp.ones((BQ, 1), jnp.float32),
             jnp.zeros((BQ, HD_ - 1), jnp.float32)],
            axis=1).astype(jnp.bfloat16)
        v3_ref[lo:lo + BQ, :] = v3b
        pv = pv + _dot(a_d, v3b, (((1,), (0,)), ((), ())))

        sum_agg = pv[:, :HD_]
        mean_agg = pv[:, HD_:2 * HD_] * dinv
        mean_sq = pv[:, 2 * HD_:3 * HD_] * dinv
        var_agg = jnp.maximum(mean_sq - mean_agg * mean_agg, 0.0)

        m = vb
        step = 1
        while step < BQ:
            pad = jnp.full((step, HD_), NEG_INF, jnp.float32)
            m = jnp.maximum(m, jnp.concatenate([pad, m[:-step]], axis=0))
            step *= 2
        m = jnp.maximum(m, carry)
        carry = m[BQ - 1:BQ, :]

        aggs.append(jnp.concatenate([sum_agg, mean_agg, m, var_agg], axis=1))

    agg = jnp.concatenate(aggs, axis=0)
    h1 = _dot(agg, w1_ref[0], (((1,), (0,)), ((), ())))
    h1 = h1 * jax.nn.sigmoid(h1)
    oh = _dot(h1, w2_ref[0], (((1,), (0,)), ((), ())))
    y = _dot(oh, wo_ref[0], (((1,), (0,)), ((), ())))

    @pl.when(h == 0)
    def _first():
        o_ref[...] = e * x_ref[...] + (1.0 - e) * y

    @pl.when(h > 0)
    def _rest():
        o_ref[...] += (1.0 - e) * y


def kernel(hidden_states, position_ids, Wq, Wk, Wv, Wo, W1, W2, eps):
    x = hidden_states.reshape(S_, D_)
    xb = x.astype(jnp.bfloat16)
    pos = position_ids.reshape(S_, 1)
    wqk = jnp.concatenate([
        Wq.reshape(D_, H_, HD_).transpose(1, 0, 2),
        Wk.reshape(D_, H_, HD_).transpose(1, 0, 2),
    ], axis=2)
    wv3 = Wv.reshape(D_, H_, HD_).transpose(1, 0, 2).astype(jnp.bfloat16)
    wo3 = Wo.reshape(H_, HD_, D_).astype(jnp.bfloat16)
    w1b = W1.astype(jnp.bfloat16)
    w2b = W2.astype(jnp.bfloat16)

    out = pl.pallas_call(
        _mega_kernel,
        grid=(H_,),
        in_specs=[
            pl.BlockSpec((S_, D_), lambda h: (0, 0)),
            pl.BlockSpec((S_, D_), lambda h: (0, 0)),
            pl.BlockSpec((S_, 1), lambda h: (0, 0)),
            pl.BlockSpec((1, D_, 2 * HD_), lambda h: (h, 0, 0)),
            pl.BlockSpec((1, D_, HD_), lambda h: (h, 0, 0)),
            pl.BlockSpec((1, AGG_IN_, MLP_HID_), lambda h: (h, 0, 0)),
            pl.BlockSpec((1, MLP_HID_, HD_), lambda h: (h, 0, 0)),
            pl.BlockSpec((1, HD_, D_), lambda h: (h, 0, 0)),
            pl.BlockSpec(memory_space=pltpu.SMEM),
        ],
        out_specs=pl.BlockSpec((S_, D_), lambda h: (0, 0)),
        out_shape=jax.ShapeDtypeStruct((S_, D_), jnp.float32),
        scratch_shapes=[
            pltpu.VMEM((S_, VW), jnp.bfloat16),
            pltpu.VMEM((S_, HD_ // 2), jnp.float32),
            pltpu.VMEM((S_, HD_ // 2), jnp.float32),
        ],
    )(x, xb, pos, wqk, wv3, w1b, w2b, wo3, jnp.reshape(eps, (1,)))

    return out.reshape(1, S_, D_)


# speedup vs baseline: 10.3068x; 1.1453x over previous
import jax
import jax.numpy as jnp
from jax.experimental import pallas as pl
from jax.experimental.pallas import tpu as pltpu

S_, D_ = 2048, 768
H_, HD_ = 12, 64
MLP_HID_ = 128
AGG_IN_ = 4 * HD_
THETA = 10000.0
NEG_INF = -3.0e38

BQ = 256
NQ = S_ // BQ
VW = 4 * HD_


def _dot(a, b, dims):
    return jax.lax.dot_general(a.astype(jnp.bfloat16), b.astype(jnp.bfloat16),
                               dims, preferred_element_type=jnp.float32)


def _mega_kernel(x_ref, xb_ref, pos_ref, wqk_ref, wv_ref, w1_ref, w2_ref,
                 wo_ref, eps_ref, o_ref, v3_ref, cos_ref, sin_ref, oh_ref):
    h = pl.program_id(0)

    @pl.when(h == 0)
    def _trig():
        pos = pos_ref[...].astype(jnp.float32)
        i = jax.lax.broadcasted_iota(
            jnp.int32, (1, HD_ // 2), 1).astype(jnp.float32)
        inv_freq = jnp.exp(i * (-2.0 * jnp.log(THETA) / HD_))
        ang = pos * inv_freq
        c = jnp.cos(ang)
        s = jnp.sin(ang)
        cos_ref[...] = jnp.concatenate([c, c], axis=1)
        sin_ref[...] = jnp.concatenate([-s, s], axis=1)

    cos_t = cos_ref[...]
    sin_t = sin_ref[...]

    qk = jax.lax.dot_general(x_ref[...], wqk_ref[0], (((1,), (0,)), ((), ())),
                             preferred_element_type=jnp.float32)
    vh = jax.lax.dot_general(xb_ref[...], wv_ref[0], (((1,), (0,)), ((), ())),
                             preferred_element_type=jnp.float32)

    def rope(t):
        rot = jnp.concatenate([t[:, HD_ // 2:], t[:, :HD_ // 2]], axis=1)
        return t * cos_t + rot * sin_t

    qh = rope(qk[:, :HD_]) * 0.125
    kh = rope(qk[:, HD_:])

    mx = vh
    step = 1
    while step < S_:
        pad = jnp.full((step, HD_), NEG_INF, jnp.float32)
        mx = jnp.maximum(mx, jnp.concatenate([pad, mx[:-step]], axis=0))
        step *= 2

    tri = (jax.lax.broadcasted_iota(jnp.int32, (BQ, BQ), 0)
           >= jax.lax.broadcasted_iota(jnp.int32, (BQ, BQ), 1))
    aggs = []
    for qi in range(NQ):
        lo = qi * BQ
        qb = qh[lo:lo + BQ, :]
        pv = jnp.zeros((BQ, VW), jnp.float32)
        for t in range(qi):
            kc = kh[t * BQ:(t + 1) * BQ, :]
            a_t = jnp.maximum(
                jax.lax.dot_general(qb, kc, (((1,), (1,)), ((), ())),
                                    preferred_element_type=jnp.float32), 0.0)
            pv = pv + _dot(a_t, v3_ref[t * BQ:(t + 1) * BQ, :],
                           (((1,), (0,)), ((), ())))
        deg = pv[:, 3 * HD_:3 * HD_ + 1]
        kd = kh[lo:lo + BQ, :]
        s_d = jax.lax.dot_general(qb, kd, (((1,), (1,)), ((), ())),
                                  preferred_element_type=jnp.float32)
        a_d = jnp.where(tri, jnp.maximum(s_d, 0.0), 0.0)
        deg = deg + jnp.sum(a_d, axis=1, keepdims=True)
        dinv = jnp.where(deg > 0.0, jax.lax.rsqrt(deg), 0.0)

        vb = vh[lo:lo + BQ, :]
        v3b = jnp.concatenate(
            [vb, vb * dinv, vb * vb * dinv,
             jnp.ones((BQ, 1), jnp.float32),
             jnp.zeros((BQ, HD_ - 1), jnp.float32)],
            axis=1).astype(jnp.bfloat16)
        v3_ref[lo:lo + BQ, :] = v3b
        pv = pv + _dot(a_d, v3b, (((1,), (0,)), ((), ())))

        sum_agg = pv[:, :HD_]
        mean_agg = pv[:, HD_:2 * HD_] * dinv
        mean_sq = pv[:, 2 * HD_:3 * HD_] * dinv
        var_agg = jnp.maximum(mean_sq - mean_agg * mean_agg, 0.0)

        aggs.append(jnp.concatenate(
            [sum_agg, mean_agg, mx[lo:lo + BQ, :], var_agg], axis=1))

    agg = jnp.concatenate(aggs, axis=0)
    h1 = _dot(agg, w1_ref[0], (((1,), (0,)), ((), ())))
    h1 = h1 * jax.nn.sigmoid(h1)
    oh = _dot(h1, w2_ref[0], (((1,), (0,)), ((), ())))
    oh_ref[h] = oh.astype(jnp.bfloat16)

    @pl.when(h == H_ - 1)
    def _final():
        e = eps_ref[0]
        y = jnp.zeros((S_, D_), jnp.float32)
        for hh in range(H_):
            y = y + jax.lax.dot_general(
                oh_ref[hh], wo_ref[hh], (((1,), (0,)), ((), ())),
                preferred_element_type=jnp.float32)
        o_ref[...] = e * x_ref[...] + (1.0 - e) * y


def kernel(hidden_states, position_ids, Wq, Wk, Wv, Wo, W1, W2, eps):
    x = hidden_states.reshape(S_, D_)
    xb = x.astype(jnp.bfloat16)
    pos = position_ids.reshape(S_, 1)
    wqk = jnp.concatenate([
        Wq.reshape(D_, H_, HD_).transpose(1, 0, 2),
        Wk.reshape(D_, H_, HD_).transpose(1, 0, 2),
    ], axis=2)
    wv3 = Wv.reshape(D_, H_, HD_).transpose(1, 0, 2).astype(jnp.bfloat16)
    wob = Wo.reshape(H_, HD_, D_).astype(jnp.bfloat16)
    w1b = W1.astype(jnp.bfloat16)
    w2b = W2.astype(jnp.bfloat16)

    out = pl.pallas_call(
        _mega_kernel,
        grid=(H_,),
        in_specs=[
            pl.BlockSpec((S_, D_), lambda h: (0, 0)),
            pl.BlockSpec((S_, D_), lambda h: (0, 0)),
            pl.BlockSpec((S_, 1), lambda h: (0, 0)),
            pl.BlockSpec((1, D_, 2 * HD_), lambda h: (h, 0, 0)),
            pl.BlockSpec((1, D_, HD_), lambda h: (h, 0, 0)),
            pl.BlockSpec((1, AGG_IN_, MLP_HID_), lambda h: (h, 0, 0)),
            pl.BlockSpec((1, MLP_HID_, HD_), lambda h: (h, 0, 0)),
            pl.BlockSpec((H_, HD_, D_), lambda h: (0, 0, 0)),
            pl.BlockSpec(memory_space=pltpu.SMEM),
        ],
        out_specs=pl.BlockSpec((S_, D_), lambda h: (0, 0)),
        out_shape=jax.ShapeDtypeStruct((S_, D_), jnp.float32),
        scratch_shapes=[
            pltpu.VMEM((S_, VW), jnp.bfloat16),
            pltpu.VMEM((S_, HD_), jnp.float32),
            pltpu.VMEM((S_, HD_), jnp.float32),
            pltpu.VMEM((H_, S_, HD_), jnp.bfloat16),
        ],
    )(x, xb, pos, wqk, wv3, w1b, w2b, wob, jnp.reshape(eps, (1,)))

    return out.reshape(1, S_, D_)


# speedup vs baseline: 11.3651x; 1.1027x over previous
import jax
import jax.numpy as jnp
from jax.experimental import pallas as pl
from jax.experimental.pallas import tpu as pltpu

S_, D_ = 2048, 768
H_, HD_ = 12, 64
MLP_HID_ = 128
AGG_IN_ = 4 * HD_
THETA = 10000.0
NEG_INF = -3.0e38

BQ = 512
NQ = S_ // BQ
VW = 4 * HD_


def _dot(a, b, dims):
    return jax.lax.dot_general(a.astype(jnp.bfloat16), b.astype(jnp.bfloat16),
                               dims, preferred_element_type=jnp.float32)


def _mega_kernel(x_ref, xb_ref, pos_ref, wqk_ref, wv_ref, w1_ref, w2_ref,
                 wo_ref, eps_ref, o_ref, v3_ref, cos_ref, sin_ref, oh_ref):
    h = pl.program_id(0)

    @pl.when(h == 0)
    def _trig():
        pos = pos_ref[...].astype(jnp.float32)
        i = jax.lax.broadcasted_iota(
            jnp.int32, (1, HD_ // 2), 1).astype(jnp.float32)
        inv_freq = jnp.exp(i * (-2.0 * jnp.log(THETA) / HD_))
        ang = pos * inv_freq
        c = jnp.cos(ang)
        s = jnp.sin(ang)
        cos_ref[...] = jnp.concatenate([c, c], axis=1)
        sin_ref[...] = jnp.concatenate([-s, s], axis=1)

    cos_t = cos_ref[...]
    sin_t = sin_ref[...]

    qk = jax.lax.dot_general(x_ref[...], wqk_ref[0], (((1,), (0,)), ((), ())),
                             preferred_element_type=jnp.float32)
    vh = jax.lax.dot_general(xb_ref[...], wv_ref[0], (((1,), (0,)), ((), ())),
                             preferred_element_type=jnp.float32)

    def rope(t):
        rot = jnp.concatenate([t[:, HD_ // 2:], t[:, :HD_ // 2]], axis=1)
        return t * cos_t + rot * sin_t

    qh = rope(qk[:, :HD_]) * 0.125
    kh = rope(qk[:, HD_:])

    mx = vh
    step = 1
    while step < S_:
        pad = jnp.full((step, HD_), NEG_INF, jnp.float32)
        mx = jnp.maximum(mx, jnp.concatenate([pad, mx[:-step]], axis=0))
        step *= 2

    tri = (jax.lax.broadcasted_iota(jnp.int32, (BQ, BQ), 0)
           >= jax.lax.broadcasted_iota(jnp.int32, (BQ, BQ), 1))
    aggs = []
    for qi in range(NQ):
        lo = qi * BQ
        qb = qh[lo:lo + BQ, :]
        pv = jnp.zeros((BQ, VW), jnp.float32)
        for t in range(qi):
            kc = kh[t * BQ:(t + 1) * BQ, :]
            a_t = jnp.maximum(
                jax.lax.dot_general(qb, kc, (((1,), (1,)), ((), ())),
                                    preferred_element_type=jnp.float32), 0.0)
            pv = pv + _dot(a_t, v3_ref[t * BQ:(t + 1) * BQ, :],
                           (((1,), (0,)), ((), ())))
        deg = pv[:, 3 * HD_:3 * HD_ + 1]
        kd = kh[lo:lo + BQ, :]
        s_d = jax.lax.dot_general(qb, kd, (((1,), (1,)), ((), ())),
                                  preferred_element_type=jnp.float32)
        a_d = jnp.where(tri, jnp.maximum(s_d, 0.0), 0.0)
        deg = deg + jnp.sum(a_d, axis=1, keepdims=True)
        dinv = jnp.where(deg > 0.0, jax.lax.rsqrt(deg), 0.0)

        vb = vh[lo:lo + BQ, :]
        v3b = jnp.concatenate(
            [vb, vb * dinv, vb * vb * dinv,
             jnp.ones((BQ, 1), jnp.float32),
             jnp.zeros((BQ, HD_ - 1), jnp.float32)],
            axis=1).astype(jnp.bfloat16)
        v3_ref[lo:lo + BQ, :] = v3b
        pv = pv + _dot(a_d, v3b, (((1,), (0,)), ((), ())))

        sum_agg = pv[:, :HD_]
        mean_agg = pv[:, HD_:2 * HD_] * dinv
        mean_sq = pv[:, 2 * HD_:3 * HD_] * dinv
        var_agg = jnp.maximum(mean_sq - mean_agg * mean_agg, 0.0)

        aggs.append(jnp.concatenate(
            [sum_agg, mean_agg, mx[lo:lo + BQ, :], var_agg], axis=1))

    agg = jnp.concatenate(aggs, axis=0)
    h1 = _dot(agg, w1_ref[0], (((1,), (0,)), ((), ())))
    h1 = h1 * jax.nn.sigmoid(h1)
    oh = _dot(h1, w2_ref[0], (((1,), (0,)), ((), ())))
    oh_ref[h] = oh.astype(jnp.bfloat16)

    @pl.when(h == H_ - 1)
    def _final():
        e = eps_ref[0]
        y = jnp.zeros((S_, D_), jnp.float32)
        for hh in range(H_):
            y = y + jax.lax.dot_general(
                oh_ref[hh], wo_ref[hh], (((1,), (0,)), ((), ())),
                preferred_element_type=jnp.float32)
        o_ref[...] = e * x_ref[...] + (1.0 - e) * y


def kernel(hidden_states, position_ids, Wq, Wk, Wv, Wo, W1, W2, eps):
    x = hidden_states.reshape(S_, D_)
    xb = x.astype(jnp.bfloat16)
    pos = position_ids.reshape(S_, 1)
    wqk = jnp.concatenate([
        Wq.reshape(D_, H_, HD_).transpose(1, 0, 2),
        Wk.reshape(D_, H_, HD_).transpose(1, 0, 2),
    ], axis=2)
    wv3 = Wv.reshape(D_, H_, HD_).transpose(1, 0, 2).astype(jnp.bfloat16)
    wob = Wo.reshape(H_, HD_, D_).astype(jnp.bfloat16)
    w1b = W1.astype(jnp.bfloat16)
    w2b = W2.astype(jnp.bfloat16)

    out = pl.pallas_call(
        _mega_kernel,
        grid=(H_,),
        in_specs=[
            pl.BlockSpec((S_, D_), lambda h: (0, 0)),
            pl.BlockSpec((S_, D_), lambda h: (0, 0)),
            pl.BlockSpec((S_, 1), lambda h: (0, 0)),
            pl.BlockSpec((1, D_, 2 * HD_), lambda h: (h, 0, 0)),
            pl.BlockSpec((1, D_, HD_), lambda h: (h, 0, 0)),
            pl.BlockSpec((1, AGG_IN_, MLP_HID_), lambda h: (h, 0, 0)),
            pl.BlockSpec((1, MLP_HID_, HD_), lambda h: (h, 0, 0)),
            pl.BlockSpec((H_, HD_, D_), lambda h: (0, 0, 0)),
            pl.BlockSpec(memory_space=pltpu.SMEM),
        ],
        out_specs=pl.BlockSpec((S_, D_), lambda h: (0, 0)),
        out_shape=jax.ShapeDtypeStruct((S_, D_), jnp.float32),
        scratch_shapes=[
            pltpu.VMEM((S_, VW), jnp.bfloat16),
            pltpu.VMEM((S_, HD_), jnp.float32),
            pltpu.VMEM((S_, HD_), jnp.float32),
            pltpu.VMEM((H_, S_, HD_), jnp.bfloat16),
        ],
    )(x, xb, pos, wqk, wv3, w1b, w2b, wob, jnp.reshape(eps, (1,)))

    return out.reshape(1, S_, D_)


# speedup vs baseline: 11.4857x; 1.0106x over previous
import jax
import jax.numpy as jnp
from jax.experimental import pallas as pl
from jax.experimental.pallas import tpu as pltpu

S_, D_ = 2048, 768
H_, HD_ = 12, 64
MLP_HID_ = 128
AGG_IN_ = 4 * HD_
THETA = 10000.0
NEG_INF = -3.0e38

BQ = 1024
NQ = S_ // BQ
VW = 4 * HD_


def _dot(a, b, dims):
    return jax.lax.dot_general(a.astype(jnp.bfloat16), b.astype(jnp.bfloat16),
                               dims, preferred_element_type=jnp.float32)


def _mega_kernel(x_ref, xb_ref, pos_ref, wqk_ref, wv_ref, w1_ref, w2_ref,
                 wo_ref, eps_ref, o_ref, v3_ref, cos_ref, sin_ref, oh_ref):
    h = pl.program_id(0)

    @pl.when(h == 0)
    def _trig():
        pos = pos_ref[...].astype(jnp.float32)
        i = jax.lax.broadcasted_iota(
            jnp.int32, (1, HD_ // 2), 1).astype(jnp.float32)
        inv_freq = jnp.exp(i * (-2.0 * jnp.log(THETA) / HD_))
        ang = pos * inv_freq
        c = jnp.cos(ang)
        s = jnp.sin(ang)
        cos_ref[...] = jnp.concatenate([c, c], axis=1)
        sin_ref[...] = jnp.concatenate([-s, s], axis=1)

    cos_t = cos_ref[...]
    sin_t = sin_ref[...]

    qk = jax.lax.dot_general(x_ref[...], wqk_ref[0], (((1,), (0,)), ((), ())),
                             preferred_element_type=jnp.float32)
    vh = jax.lax.dot_general(xb_ref[...], wv_ref[0], (((1,), (0,)), ((), ())),
                             preferred_element_type=jnp.float32)

    def rope(t):
        rot = jnp.concatenate([t[:, HD_ // 2:], t[:, :HD_ // 2]], axis=1)
        return t * cos_t + rot * sin_t

    qh = rope(qk[:, :HD_]) * 0.125
    kh = rope(qk[:, HD_:])

    mx = vh
    step = 1
    while step < S_:
        pad = jnp.full((step, HD_), NEG_INF, jnp.float32)
        mx = jnp.maximum(mx, jnp.concatenate([pad, mx[:-step]], axis=0))
        step *= 2

    tri = (jax.lax.broadcasted_iota(jnp.int32, (BQ, BQ), 0)
           >= jax.lax.broadcasted_iota(jnp.int32, (BQ, BQ), 1))
    aggs = []
    for qi in range(NQ):
        lo = qi * BQ
        qb = qh[lo:lo + BQ, :]
        pv = jnp.zeros((BQ, VW), jnp.float32)
        for t in range(qi):
            kc = kh[t * BQ:(t + 1) * BQ, :]
            a_t = jnp.maximum(
                jax.lax.dot_general(qb, kc, (((1,), (1,)), ((), ())),
                                    preferred_element_type=jnp.float32), 0.0)
            pv = pv + _dot(a_t, v3_ref[t * BQ:(t + 1) * BQ, :],
                           (((1,), (0,)), ((), ())))
        deg = pv[:, 3 * HD_:3 * HD_ + 1]
        kd = kh[lo:lo + BQ, :]
        s_d = jax.lax.dot_general(qb, kd, (((1,), (1,)), ((), ())),
                                  preferred_element_type=jnp.float32)
        a_d = jnp.where(tri, jnp.maximum(s_d, 0.0), 0.0)
        deg = deg + jnp.sum(a_d, axis=1, keepdims=True)
        dinv = jnp.where(deg > 0.0, jax.lax.rsqrt(deg), 0.0)

        vb = vh[lo:lo + BQ, :]
        v3b = jnp.concatenate(
            [vb, vb * dinv, vb * vb * dinv,
             jnp.ones((BQ, 1), jnp.float32),
             jnp.zeros((BQ, HD_ - 1), jnp.float32)],
            axis=1).astype(jnp.bfloat16)
        v3_ref[lo:lo + BQ, :] = v3b
        pv = pv + _dot(a_d, v3b, (((1,), (0,)), ((), ())))

        sum_agg = pv[:, :HD_]
        mean_agg = pv[:, HD_:2 * HD_] * dinv
        mean_sq = pv[:, 2 * HD_:3 * HD_] * dinv
        var_agg = jnp.maximum(mean_sq - mean_agg * mean_agg, 0.0)

        aggs.append(jnp.concatenate(
            [sum_agg, mean_agg, mx[lo:lo + BQ, :], var_agg], axis=1))

    agg = jnp.concatenate(aggs, axis=0)
    h1 = _dot(agg, w1_ref[0], (((1,), (0,)), ((), ())))
    h1 = h1 * jax.nn.sigmoid(h1)
    oh = _dot(h1, w2_ref[0], (((1,), (0,)), ((), ())))
    oh_ref[h] = oh.astype(jnp.bfloat16)

    @pl.when(h == H_ - 1)
    def _final():
        e = eps_ref[0]
        y = jnp.zeros((S_, D_), jnp.float32)
        for hh in range(H_):
            y = y + jax.lax.dot_general(
                oh_ref[hh], wo_ref[hh], (((1,), (0,)), ((), ())),
                preferred_element_type=jnp.float32)
        o_ref[...] = e * x_ref[...] + (1.0 - e) * y


def kernel(hidden_states, position_ids, Wq, Wk, Wv, Wo, W1, W2, eps):
    x = hidden_states.reshape(S_, D_)
    xb = x.astype(jnp.bfloat16)
    pos = position_ids.reshape(S_, 1)
    wqk = jnp.concatenate([
        Wq.reshape(D_, H_, HD_).transpose(1, 0, 2),
        Wk.reshape(D_, H_, HD_).transpose(1, 0, 2),
    ], axis=2)
    wv3 = Wv.reshape(D_, H_, HD_).transpose(1, 0, 2).astype(jnp.bfloat16)
    wob = Wo.reshape(H_, HD_, D_).astype(jnp.bfloat16)
    w1b = W1.astype(jnp.bfloat16)
    w2b = W2.astype(jnp.bfloat16)

    out = pl.pallas_call(
        _mega_kernel,
        grid=(H_,),
        in_specs=[
            pl.BlockSpec((S_, D_), lambda h: (0, 0)),
            pl.BlockSpec((S_, D_), lambda h: (0, 0)),
            pl.BlockSpec((S_, 1), lambda h: (0, 0)),
            pl.BlockSpec((1, D_, 2 * HD_), lambda h: (h, 0, 0)),
            pl.BlockSpec((1, D_, HD_), lambda h: (h, 0, 0)),
            pl.BlockSpec((1, AGG_IN_, MLP_HID_), lambda h: (h, 0, 0)),
            pl.BlockSpec((1, MLP_HID_, HD_), lambda h: (h, 0, 0)),
            pl.BlockSpec((H_, HD_, D_), lambda h: (0, 0, 0)),
            pl.BlockSpec(memory_space=pltpu.SMEM),
        ],
        out_specs=pl.BlockSpec((S_, D_), lambda h: (0, 0)),
        out_shape=jax.ShapeDtypeStruct((S_, D_), jnp.float32),
        scratch_shapes=[
            pltpu.VMEM((S_, VW), jnp.bfloat16),
            pltpu.VMEM((S_, HD_), jnp.float32),
            pltpu.VMEM((S_, HD_), jnp.float32),
            pltpu.VMEM((H_, S_, HD_), jnp.bfloat16),
        ],
    )(x, xb, pos, wqk, wv3, w1b, w2b, wob, jnp.reshape(eps, (1,)))

    return out.reshape(1, S_, D_)


# speedup vs baseline: 11.7724x; 1.0250x over previous
import jax
import jax.numpy as jnp
from jax.experimental import pallas as pl
from jax.experimental.pallas import tpu as pltpu

S_, D_ = 2048, 768
H_, HD_ = 12, 64
MLP_HID_ = 128
AGG_IN_ = 4 * HD_
THETA = 10000.0
NEG_INF = -3.0e38

BQ = 1024
NQ = S_ // BQ
VW = 4 * HD_


def _dot(a, b, dims):
    return jax.lax.dot_general(a.astype(jnp.bfloat16), b.astype(jnp.bfloat16),
                               dims, preferred_element_type=jnp.float32)


def _mega_kernel(x_ref, xb_ref, cos_ref, sin_ref, wqk_ref, wv_ref, w1_ref,
                 w2_ref, wo_ref, eps_ref, o_ref, v3_ref, oh_ref):
    h = pl.program_id(0)
    cos_t = cos_ref[...]
    sin_t = sin_ref[...]

    qk = jax.lax.dot_general(x_ref[...], wqk_ref[0], (((1,), (0,)), ((), ())),
                             preferred_element_type=jnp.float32)
    vh = jax.lax.dot_general(xb_ref[...], wv_ref[0], (((1,), (0,)), ((), ())),
                             preferred_element_type=jnp.float32)

    def rope(t):
        rot = jnp.concatenate([t[:, HD_ // 2:], t[:, :HD_ // 2]], axis=1)
        return t * cos_t + rot * sin_t

    qh = rope(qk[:, :HD_]) * 0.125
    kh = rope(qk[:, HD_:])

    mx = vh
    step = 1
    while step < S_:
        pad = jnp.full((step, HD_), NEG_INF, jnp.float32)
        mx = jnp.maximum(mx, jnp.concatenate([pad, mx[:-step]], axis=0))
        step *= 2

    tri = (jax.lax.broadcasted_iota(jnp.int32, (BQ, BQ), 0)
           >= jax.lax.broadcasted_iota(jnp.int32, (BQ, BQ), 1))
    aggs = []
    for qi in range(NQ):
        lo = qi * BQ
        qb = qh[lo:lo + BQ, :]
        pv = jnp.zeros((BQ, VW), jnp.float32)
        for t in range(qi):
            kc = kh[t * BQ:(t + 1) * BQ, :]
            a_t = jnp.maximum(
                jax.lax.dot_general(qb, kc, (((1,), (1,)), ((), ())),
                                    preferred_element_type=jnp.float32), 0.0)
            pv = pv + _dot(a_t, v3_ref[t * BQ:(t + 1) * BQ, :],
                           (((1,), (0,)), ((), ())))
        deg = pv[:, 3 * HD_:3 * HD_ + 1]
        kd = kh[lo:lo + BQ, :]
        s_d = jax.lax.dot_general(qb, kd, (((1,), (1,)), ((), ())),
                                  preferred_element_type=jnp.float32)
        a_d = jnp.where(tri, jnp.maximum(s_d, 0.0), 0.0)
        deg = deg + jnp.sum(a_d, axis=1, keepdims=True)
        dinv = jnp.where(deg > 0.0, jax.lax.rsqrt(deg), 0.0)

        vb = vh[lo:lo + BQ, :]
        v3b = jnp.concatenate(
            [vb, vb * dinv, vb * vb * dinv,
             jnp.ones((BQ, 1), jnp.float32),
             jnp.zeros((BQ, HD_ - 1), jnp.float32)],
            axis=1).astype(jnp.bfloat16)
        v3_ref[lo:lo + BQ, :] = v3b
        pv = pv + _dot(a_d, v3b, (((1,), (0,)), ((), ())))

        sum_agg = pv[:, :HD_]
        mean_agg = pv[:, HD_:2 * HD_] * dinv
        mean_sq = pv[:, 2 * HD_:3 * HD_] * dinv
        var_agg = jnp.maximum(mean_sq - mean_agg * mean_agg, 0.0)

        aggs.append(jnp.concatenate(
            [sum_agg, mean_agg, mx[lo:lo + BQ, :], var_agg], axis=1))

    agg = jnp.concatenate(aggs, axis=0)
    h1 = _dot(agg, w1_ref[0], (((1,), (0,)), ((), ())))
    h1 = h1 * jax.nn.sigmoid(h1)
    oh = _dot(h1, w2_ref[0], (((1,), (0,)), ((), ())))
    oh_ref[h] = oh.astype(jnp.bfloat16)

    @pl.when(h == H_ - 1)
    def _final():
        e = eps_ref[0]
        oh_all = jnp.concatenate(
            [oh_ref[hh] for hh in range(H_)], axis=1)
        y = jax.lax.dot_general(
            oh_all, wo_ref[...], (((1,), (0,)), ((), ())),
            preferred_element_type=jnp.float32)
        o_ref[...] = e * x_ref[...] + (1.0 - e) * y


def kernel(hidden_states, position_ids, Wq, Wk, Wv, Wo, W1, W2, eps):
    x = hidden_states.reshape(S_, D_)
    xb = x.astype(jnp.bfloat16)
    pos = position_ids.reshape(S_, 1).astype(jnp.float32)
    inv_freq = jnp.exp(jnp.arange(HD_ // 2, dtype=jnp.float32)
                       * (-2.0 * jnp.log(THETA) / HD_))
    ang = pos * inv_freq[None, :]
    cos_t = jnp.concatenate([jnp.cos(ang)] * 2, axis=1)
    sin_t = jnp.concatenate([-jnp.sin(ang), jnp.sin(ang)], axis=1)
    wqk = jnp.concatenate([
        Wq.reshape(D_, H_, HD_).transpose(1, 0, 2),
        Wk.reshape(D_, H_, HD_).transpose(1, 0, 2),
    ], axis=2)
    wv3 = Wv.reshape(D_, H_, HD_).transpose(1, 0, 2).astype(jnp.bfloat16)
    wob = Wo.astype(jnp.bfloat16)
    w1b = W1.astype(jnp.bfloat16)
    w2b = W2.astype(jnp.bfloat16)

    out = pl.pallas_call(
        _mega_kernel,
        grid=(H_,),
        in_specs=[
            pl.BlockSpec((S_, D_), lambda h: (0, 0)),
            pl.BlockSpec((S_, D_), lambda h: (0, 0)),
            pl.BlockSpec((S_, HD_), lambda h: (0, 0)),
            pl.BlockSpec((S_, HD_), lambda h: (0, 0)),
            pl.BlockSpec((1, D_, 2 * HD_), lambda h: (h, 0, 0)),
            pl.BlockSpec((1, D_, HD_), lambda h: (h, 0, 0)),
            pl.BlockSpec((1, AGG_IN_, MLP_HID_), lambda h: (h, 0, 0)),
            pl.BlockSpec((1, MLP_HID_, HD_), lambda h: (h, 0, 0)),
            pl.BlockSpec((H_ * HD_, D_), lambda h: (0, 0)),
            pl.BlockSpec(memory_space=pltpu.SMEM),
        ],
        out_specs=pl.BlockSpec((S_, D_), lambda h: (0, 0)),
        out_shape=jax.ShapeDtypeStruct((S_, D_), jnp.float32),
        scratch_shapes=[
            pltpu.VMEM((S_, VW), jnp.bfloat16),
            pltpu.VMEM((H_, S_, HD_), jnp.bfloat16),
        ],
    )(x, xb, cos_t, sin_t, wqk, wv3, w1b, w2b, wob, jnp.reshape(eps, (1,)))

    return out.reshape(1, S_, D_)


# speedup vs baseline: 12.6734x; 1.0765x over previous
import jax
import jax.numpy as jnp
from jax.experimental import pallas as pl
from jax.experimental.pallas import tpu as pltpu

S_, D_ = 2048, 768
H_, HD_ = 12, 64
MLP_HID_ = 128
AGG_IN_ = 4 * HD_
THETA = 10000.0
NEG_INF = -3.0e38

BQ = 1024
NQ = S_ // BQ
VW = 4 * HD_
HPG = 2
NG = H_ // HPG


def _dot(a, b, dims):
    return jax.lax.dot_general(a.astype(jnp.bfloat16), b.astype(jnp.bfloat16),
                               dims, preferred_element_type=jnp.float32)


def _mega_kernel(x_ref, xb_ref, cos_ref, sin_ref, wqk_ref, wv_ref, w1_ref,
                 w2_ref, wo_ref, eps_ref, o_ref, v3_ref, oh_ref):
    h = pl.program_id(0)
    cos_t = cos_ref[...]
    sin_t = sin_ref[...]

    qk2 = jax.lax.dot_general(x_ref[...], wqk_ref[0], (((1,), (0,)), ((), ())),
                              preferred_element_type=jnp.float32)
    vh2 = jax.lax.dot_general(xb_ref[...], wv_ref[0], (((1,), (0,)), ((), ())),
                              preferred_element_type=jnp.float32)

    def rope(t):
        rot = jnp.concatenate([t[:, HD_ // 2:], t[:, :HD_ // 2]], axis=1)
        return t * cos_t + rot * sin_t

    tri = (jax.lax.broadcasted_iota(jnp.int32, (BQ, BQ), 0)
           >= jax.lax.broadcasted_iota(jnp.int32, (BQ, BQ), 1))

    for sub in range(HPG):
        qh = rope(qk2[:, 2 * sub * HD_:(2 * sub + 1) * HD_]) * 0.125
        kh = rope(qk2[:, (2 * sub + 1) * HD_:(2 * sub + 2) * HD_])
        vh = vh2[:, sub * HD_:(sub + 1) * HD_]

        mx = vh
        step = 1
        while step < S_:
            pad = jnp.full((step, HD_), NEG_INF, jnp.float32)
            mx = jnp.maximum(mx, jnp.concatenate([pad, mx[:-step]], axis=0))
            step *= 2

        aggs = []
        for qi in range(NQ):
            lo = qi * BQ
            qb = qh[lo:lo + BQ, :]
            pv = jnp.zeros((BQ, VW), jnp.float32)
            for t in range(qi):
                kc = kh[t * BQ:(t + 1) * BQ, :]
                a_t = jnp.maximum(
                    jax.lax.dot_general(qb, kc, (((1,), (1,)), ((), ())),
                                        preferred_element_type=jnp.float32),
                    0.0)
                pv = pv + _dot(a_t, v3_ref[t * BQ:(t + 1) * BQ, :],
                               (((1,), (0,)), ((), ())))
            deg = pv[:, 3 * HD_:3 * HD_ + 1]
            kd = kh[lo:lo + BQ, :]
            s_d = jax.lax.dot_general(qb, kd, (((1,), (1,)), ((), ())),
                                      preferred_element_type=jnp.float32)
            a_d = jnp.where(tri, jnp.maximum(s_d, 0.0), 0.0)
            deg = deg + jnp.sum(a_d, axis=1, keepdims=True)
            dinv = jnp.where(deg > 0.0, jax.lax.rsqrt(deg), 0.0)

            vb = vh[lo:lo + BQ, :]
            v3b = jnp.concatenate(
                [vb, vb * dinv, vb * vb * dinv,
                 jnp.ones((BQ, 1), jnp.float32),
                 jnp.zeros((BQ, HD_ - 1), jnp.float32)],
                axis=1).astype(jnp.bfloat16)
            v3_ref[lo:lo + BQ, :] = v3b
            pv = pv + _dot(a_d, v3b, (((1,), (0,)), ((), ())))

            sum_agg = pv[:, :HD_]
            mean_agg = pv[:, HD_:2 * HD_] * dinv
            mean_sq = pv[:, 2 * HD_:3 * HD_] * dinv
            var_agg = jnp.maximum(mean_sq - mean_agg * mean_agg, 0.0)

            aggs.append(jnp.concatenate(
                [sum_agg, mean_agg, mx[lo:lo + BQ, :], var_agg], axis=1))

        agg = jnp.concatenate(aggs, axis=0)
        h1 = _dot(agg, w1_ref[0, sub], (((1,), (0,)), ((), ())))
        h1 = h1 * jax.nn.sigmoid(h1)
        oh = _dot(h1, w2_ref[0, sub], (((1,), (0,)), ((), ())))
        oh_ref[HPG * h + sub] = oh.astype(jnp.bfloat16)

    @pl.when(h == H_ // HPG - 1)
    def _final():
        e = eps_ref[0]
        oh_all = jnp.concatenate(
            [oh_ref[hh] for hh in range(H_)], axis=1)
        y = jax.lax.dot_general(
            oh_all, wo_ref[...], (((1,), (0,)), ((), ())),
            preferred_element_type=jnp.float32)
        o_ref[...] = e * x_ref[...] + (1.0 - e) * y


def kernel(hidden_states, position_ids, Wq, Wk, Wv, Wo, W1, W2, eps):
    x = hidden_states.reshape(S_, D_)
    xb = x.astype(jnp.bfloat16)
    pos = position_ids.reshape(S_, 1).astype(jnp.float32)
    inv_freq = jnp.exp(jnp.arange(HD_ // 2, dtype=jnp.float32)
                       * (-2.0 * jnp.log(THETA) / HD_))
    ang = pos * inv_freq[None, :]
    cos_t = jnp.concatenate([jnp.cos(ang)] * 2, axis=1)
    sin_t = jnp.concatenate([-jnp.sin(ang), jnp.sin(ang)], axis=1)
    wqk = jnp.concatenate([
        Wq.reshape(D_, H_, HD_).transpose(1, 0, 2),
        Wk.reshape(D_, H_, HD_).transpose(1, 0, 2),
    ], axis=2)
    wqk = (wqk.reshape(NG, HPG, D_, 2 * HD_)
           .transpose(0, 2, 1, 3).reshape(NG, D_, HPG * 2 * HD_))
    wv3 = (Wv.reshape(D_, H_, HD_).transpose(1, 0, 2)
           .reshape(NG, HPG, D_, HD_).transpose(0, 2, 1, 3)
           .reshape(NG, D_, HPG * HD_).astype(jnp.bfloat16))
    wob = Wo.astype(jnp.bfloat16)
    w1b = W1.reshape(NG, HPG, AGG_IN_, MLP_HID_).astype(jnp.bfloat16)
    w2b = W2.reshape(NG, HPG, MLP_HID_, HD_).astype(jnp.bfloat16)

    out = pl.pallas_call(
        _mega_kernel,
        grid=(NG,),
        in_specs=[
            pl.BlockSpec((S_, D_), lambda h: (0, 0)),
            pl.BlockSpec((S_, D_), lambda h: (0, 0)),
            pl.BlockSpec((S_, HD_), lambda h: (0, 0)),
            pl.BlockSpec((S_, HD_), lambda h: (0, 0)),
            pl.BlockSpec((1, D_, HPG * 2 * HD_), lambda h: (h, 0, 0)),
            pl.BlockSpec((1, D_, HPG * HD_), lambda h: (h, 0, 0)),
            pl.BlockSpec((1, HPG, AGG_IN_, MLP_HID_), lambda h: (h, 0, 0, 0)),
            pl.BlockSpec((1, HPG, MLP_HID_, HD_), lambda h: (h, 0, 0, 0)),
            pl.BlockSpec((H_ * HD_, D_), lambda h: (0, 0)),
            pl.BlockSpec(memory_space=pltpu.SMEM),
        ],
        out_specs=pl.BlockSpec((S_, D_), lambda h: (0, 0)),
        out_shape=jax.ShapeDtypeStruct((S_, D_), jnp.float32),
        scratch_shapes=[
            pltpu.VMEM((S_, VW), jnp.bfloat16),
            pltpu.VMEM((H_, S_, HD_), jnp.bfloat16),
        ],
    )(x, xb, cos_t, sin_t, wqk, wv3, w1b, w2b, wob, jnp.reshape(eps, (1,)))

    return out.reshape(1, S_, D_)


# speedup vs baseline: 13.0677x; 1.0311x over previous
import jax
import jax.numpy as jnp
from jax.experimental import pallas as pl
from jax.experimental.pallas import tpu as pltpu

S_, D_ = 2048, 768
H_, HD_ = 12, 64
MLP_HID_ = 128
AGG_IN_ = 4 * HD_
THETA = 10000.0
NEG_INF = -3.0e38

BQ = 1024
NQ = S_ // BQ
VW = 4 * HD_
HPG = 3
NG = H_ // HPG


def _dot(a, b, dims):
    return jax.lax.dot_general(a.astype(jnp.bfloat16), b.astype(jnp.bfloat16),
                               dims, preferred_element_type=jnp.float32)


def _mega_kernel(x_ref, xb_ref, cos_ref, sin_ref, wqk_ref, wv_ref, w1_ref,
                 w2_ref, wo_ref, eps_ref, o_ref, v3_ref, oh_ref):
    h = pl.program_id(0)
    cos_t = cos_ref[...]
    sin_t = sin_ref[...]

    qk2 = jax.lax.dot_general(x_ref[...], wqk_ref[0], (((1,), (0,)), ((), ())),
                              preferred_element_type=jnp.float32)
    vh2 = jax.lax.dot_general(xb_ref[...], wv_ref[0], (((1,), (0,)), ((), ())),
                              preferred_element_type=jnp.float32)

    def rope(t):
        rot = jnp.concatenate([t[:, HD_ // 2:], t[:, :HD_ // 2]], axis=1)
        return t * cos_t + rot * sin_t

    tri = (jax.lax.broadcasted_iota(jnp.int32, (BQ, BQ), 0)
           >= jax.lax.broadcasted_iota(jnp.int32, (BQ, BQ), 1))

    for sub in range(HPG):
        qh = rope(qk2[:, 2 * sub * HD_:(2 * sub + 1) * HD_]) * 0.125
        kh = rope(qk2[:, (2 * sub + 1) * HD_:(2 * sub + 2) * HD_])
        vh = vh2[:, sub * HD_:(sub + 1) * HD_]

        mx = vh
        step = 1
        while step < S_:
            pad = jnp.full((step, HD_), NEG_INF, jnp.float32)
            mx = jnp.maximum(mx, jnp.concatenate([pad, mx[:-step]], axis=0))
            step *= 2

        aggs = []
        for qi in range(NQ):
            lo = qi * BQ
            qb = qh[lo:lo + BQ, :]
            pv = jnp.zeros((BQ, VW), jnp.float32)
            for t in range(qi):
                kc = kh[t * BQ:(t + 1) * BQ, :]
                a_t = jnp.maximum(
                    jax.lax.dot_general(qb, kc, (((1,), (1,)), ((), ())),
                                        preferred_element_type=jnp.float32),
                    0.0)
                pv = pv + _dot(a_t, v3_ref[t * BQ:(t + 1) * BQ, :],
                               (((1,), (0,)), ((), ())))
            deg = pv[:, 3 * HD_:3 * HD_ + 1]
            kd = kh[lo:lo + BQ, :]
            s_d = jax.lax.dot_general(qb, kd, (((1,), (1,)), ((), ())),
                                      preferred_element_type=jnp.float32)
            a_d = jnp.where(tri, jnp.maximum(s_d, 0.0), 0.0)
            deg = deg + jnp.sum(a_d, axis=1, keepdims=True)
            dinv = jnp.where(deg > 0.0, jax.lax.rsqrt(deg), 0.0)

            vb = vh[lo:lo + BQ, :]
            v3b = jnp.concatenate(
                [vb, vb * dinv, vb * vb * dinv,
                 jnp.ones((BQ, 1), jnp.float32),
                 jnp.zeros((BQ, HD_ - 1), jnp.float32)],
                axis=1).astype(jnp.bfloat16)
            v3_ref[lo:lo + BQ, :] = v3b
            pv = pv + _dot(a_d, v3b, (((1,), (0,)), ((), ())))

            sum_agg = pv[:, :HD_]
            mean_agg = pv[:, HD_:2 * HD_] * dinv
            mean_sq = pv[:, 2 * HD_:3 * HD_] * dinv
            var_agg = jnp.maximum(mean_sq - mean_agg * mean_agg, 0.0)

            aggs.append(jnp.concatenate(
                [sum_agg, mean_agg, mx[lo:lo + BQ, :], var_agg], axis=1))

        agg = jnp.concatenate(aggs, axis=0)
        h1 = _dot(agg, w1_ref[0, sub], (((1,), (0,)), ((), ())))
        h1 = h1 * jax.nn.sigmoid(h1)
        oh = _dot(h1, w2_ref[0, sub], (((1,), (0,)), ((), ())))
        oh_ref[HPG * h + sub] = oh.astype(jnp.bfloat16)

    @pl.when(h == H_ // HPG - 1)
    def _final():
        e = eps_ref[0]
        oh_all = jnp.concatenate(
            [oh_ref[hh] for hh in range(H_)], axis=1)
        y = jax.lax.dot_general(
            oh_all, wo_ref[...], (((1,), (0,)), ((), ())),
            preferred_element_type=jnp.float32)
        o_ref[...] = e * x_ref[...] + (1.0 - e) * y


def kernel(hidden_states, position_ids, Wq, Wk, Wv, Wo, W1, W2, eps):
    x = hidden_states.reshape(S_, D_)
    xb = x.astype(jnp.bfloat16)
    pos = position_ids.reshape(S_, 1).astype(jnp.float32)
    inv_freq = jnp.exp(jnp.arange(HD_ // 2, dtype=jnp.float32)
                       * (-2.0 * jnp.log(THETA) / HD_))
    ang = pos * inv_freq[None, :]
    cos_t = jnp.concatenate([jnp.cos(ang)] * 2, axis=1)
    sin_t = jnp.concatenate([-jnp.sin(ang), jnp.sin(ang)], axis=1)
    wqk = jnp.concatenate([
        Wq.reshape(D_, H_, HD_).transpose(1, 0, 2),
        Wk.reshape(D_, H_, HD_).transpose(1, 0, 2),
    ], axis=2)
    wqk = (wqk.reshape(NG, HPG, D_, 2 * HD_)
           .transpose(0, 2, 1, 3).reshape(NG, D_, HPG * 2 * HD_))
    wv3 = (Wv.reshape(D_, H_, HD_).transpose(1, 0, 2)
           .reshape(NG, HPG, D_, HD_).transpose(0, 2, 1, 3)
           .reshape(NG, D_, HPG * HD_).astype(jnp.bfloat16))
    wob = Wo.astype(jnp.bfloat16)
    w1b = W1.reshape(NG, HPG, AGG_IN_, MLP_HID_).astype(jnp.bfloat16)
    w2b = W2.reshape(NG, HPG, MLP_HID_, HD_).astype(jnp.bfloat16)

    out = pl.pallas_call(
        _mega_kernel,
        grid=(NG,),
        in_specs=[
            pl.BlockSpec((S_, D_), lambda h: (0, 0)),
            pl.BlockSpec((S_, D_), lambda h: (0, 0)),
            pl.BlockSpec((S_, HD_), lambda h: (0, 0)),
            pl.BlockSpec((S_, HD_), lambda h: (0, 0)),
            pl.BlockSpec((1, D_, HPG * 2 * HD_), lambda h: (h, 0, 0)),
            pl.BlockSpec((1, D_, HPG * HD_), lambda h: (h, 0, 0)),
            pl.BlockSpec((1, HPG, AGG_IN_, MLP_HID_), lambda h: (h, 0, 0, 0)),
            pl.BlockSpec((1, HPG, MLP_HID_, HD_), lambda h: (h, 0, 0, 0)),
            pl.BlockSpec((H_ * HD_, D_), lambda h: (0, 0)),
            pl.BlockSpec(memory_space=pltpu.SMEM),
        ],
        out_specs=pl.BlockSpec((S_, D_), lambda h: (0, 0)),
        out_shape=jax.ShapeDtypeStruct((S_, D_), jnp.float32),
        scratch_shapes=[
            pltpu.VMEM((S_, VW), jnp.bfloat16),
            pltpu.VMEM((H_, S_, HD_), jnp.bfloat16),
        ],
    )(x, xb, cos_t, sin_t, wqk, wv3, w1b, w2b, wob, jnp.reshape(eps, (1,)))

    return out.reshape(1, S_, D_)


# speedup vs baseline: 13.2766x; 1.0160x over previous
import jax
import jax.numpy as jnp
from jax.experimental import pallas as pl
from jax.experimental.pallas import tpu as pltpu

S_, D_ = 2048, 768
H_, HD_ = 12, 64
MLP_HID_ = 128
AGG_IN_ = 4 * HD_
THETA = 10000.0
NEG_INF = -3.0e38

BQ = 1024
NQ = S_ // BQ
VW = 4 * HD_
HPG = 4
NG = H_ // HPG


def _dot(a, b, dims):
    return jax.lax.dot_general(a.astype(jnp.bfloat16), b.astype(jnp.bfloat16),
                               dims, preferred_element_type=jnp.float32)


def _mega_kernel(x_ref, xb_ref, cos_ref, sin_ref, wqk_ref, wv_ref, w1_ref,
                 w2_ref, wo_ref, eps_ref, o_ref, v3_ref, oh_ref):
    h = pl.program_id(0)
    cos_t = cos_ref[...]
    sin_t = sin_ref[...]

    qk2 = jax.lax.dot_general(x_ref[...], wqk_ref[0], (((1,), (0,)), ((), ())),
                              preferred_element_type=jnp.float32)
    vh2 = jax.lax.dot_general(xb_ref[...], wv_ref[0], (((1,), (0,)), ((), ())),
                              preferred_element_type=jnp.float32)

    def rope(t):
        rot = jnp.concatenate([t[:, HD_ // 2:], t[:, :HD_ // 2]], axis=1)
        return t * cos_t + rot * sin_t

    tri = (jax.lax.broadcasted_iota(jnp.int32, (BQ, BQ), 0)
           >= jax.lax.broadcasted_iota(jnp.int32, (BQ, BQ), 1))

    for sub in range(HPG):
        qh = rope(qk2[:, 2 * sub * HD_:(2 * sub + 1) * HD_]) * 0.125
        kh = rope(qk2[:, (2 * sub + 1) * HD_:(2 * sub + 2) * HD_])
        vh = vh2[:, sub * HD_:(sub + 1) * HD_]

        mx = vh
        step = 1
        while step < S_:
            pad = jnp.full((step, HD_), NEG_INF, jnp.float32)
            mx = jnp.maximum(mx, jnp.concatenate([pad, mx[:-step]], axis=0))
            step *= 2

        aggs = []
        for qi in range(NQ):
            lo = qi * BQ
            qb = qh[lo:lo + BQ, :]
            pv = jnp.zeros((BQ, VW), jnp.float32)
            for t in range(qi):
                kc = kh[t * BQ:(t + 1) * BQ, :]
                a_t = jnp.maximum(
                    jax.lax.dot_general(qb, kc, (((1,), (1,)), ((), ())),
                                        preferred_element_type=jnp.float32),
                    0.0)
                pv = pv + _dot(a_t, v3_ref[t * BQ:(t + 1) * BQ, :],
                               (((1,), (0,)), ((), ())))
            deg = pv[:, 3 * HD_:3 * HD_ + 1]
            kd = kh[lo:lo + BQ, :]
            s_d = jax.lax.dot_general(qb, kd, (((1,), (1,)), ((), ())),
                                      preferred_element_type=jnp.float32)
            a_d = jnp.where(tri, jnp.maximum(s_d, 0.0), 0.0)
            deg = deg + jnp.sum(a_d, axis=1, keepdims=True)
            dinv = jnp.where(deg > 0.0, jax.lax.rsqrt(deg), 0.0)

            vb = vh[lo:lo + BQ, :]
            v3b = jnp.concatenate(
                [vb, vb * dinv, vb * vb * dinv,
                 jnp.ones((BQ, 1), jnp.float32),
                 jnp.zeros((BQ, HD_ - 1), jnp.float32)],
                axis=1).astype(jnp.bfloat16)
            v3_ref[lo:lo + BQ, :] = v3b
            pv = pv + _dot(a_d, v3b, (((1,), (0,)), ((), ())))

            sum_agg = pv[:, :HD_]
            mean_agg = pv[:, HD_:2 * HD_] * dinv
            mean_sq = pv[:, 2 * HD_:3 * HD_] * dinv
            var_agg = jnp.maximum(mean_sq - mean_agg * mean_agg, 0.0)

            aggs.append(jnp.concatenate(
                [sum_agg, mean_agg, mx[lo:lo + BQ, :], var_agg], axis=1))

        agg = jnp.concatenate(aggs, axis=0)
        h1 = _dot(agg, w1_ref[0, sub], (((1,), (0,)), ((), ())))
        h1 = h1 * jax.nn.sigmoid(h1)
        oh = _dot(h1, w2_ref[0, sub], (((1,), (0,)), ((), ())))
        oh_ref[HPG * h + sub] = oh.astype(jnp.bfloat16)

    @pl.when(h == H_ // HPG - 1)
    def _final():
        e = eps_ref[0]
        oh_all = jnp.concatenate(
            [oh_ref[hh] for hh in range(H_)], axis=1)
        y = jax.lax.dot_general(
            oh_all, wo_ref[...], (((1,), (0,)), ((), ())),
            preferred_element_type=jnp.float32)
        o_ref[...] = e * x_ref[...] + (1.0 - e) * y


def kernel(hidden_states, position_ids, Wq, Wk, Wv, Wo, W1, W2, eps):
    x = hidden_states.reshape(S_, D_)
    xb = x.astype(jnp.bfloat16)
    pos = position_ids.reshape(S_, 1).astype(jnp.float32)
    inv_freq = jnp.exp(jnp.arange(HD_ // 2, dtype=jnp.float32)
                       * (-2.0 * jnp.log(THETA) / HD_))
    ang = pos * inv_freq[None, :]
    cos_t = jnp.concatenate([jnp.cos(ang)] * 2, axis=1)
    sin_t = jnp.concatenate([-jnp.sin(ang), jnp.sin(ang)], axis=1)
    wqk = jnp.concatenate([
        Wq.reshape(D_, H_, HD_).transpose(1, 0, 2),
        Wk.reshape(D_, H_, HD_).transpose(1, 0, 2),
    ], axis=2)
    wqk = (wqk.reshape(NG, HPG, D_, 2 * HD_)
           .transpose(0, 2, 1, 3).reshape(NG, D_, HPG * 2 * HD_))
    wv3 = (Wv.reshape(D_, H_, HD_).transpose(1, 0, 2)
           .reshape(NG, HPG, D_, HD_).transpose(0, 2, 1, 3)
           .reshape(NG, D_, HPG * HD_).astype(jnp.bfloat16))
    wob = Wo.astype(jnp.bfloat16)
    w1b = W1.reshape(NG, HPG, AGG_IN_, MLP_HID_).astype(jnp.bfloat16)
    w2b = W2.reshape(NG, HPG, MLP_HID_, HD_).astype(jnp.bfloat16)

    out = pl.pallas_call(
        _mega_kernel,
        grid=(NG,),
        in_specs=[
            pl.BlockSpec((S_, D_), lambda h: (0, 0)),
            pl.BlockSpec((S_, D_), lambda h: (0, 0)),
            pl.BlockSpec((S_, HD_), lambda h: (0, 0)),
            pl.BlockSpec((S_, HD_), lambda h: (0, 0)),
            pl.BlockSpec((1, D_, HPG * 2 * HD_), lambda h: (h, 0, 0)),
            pl.BlockSpec((1, D_, HPG * HD_), lambda h: (h, 0, 0)),
            pl.BlockSpec((1, HPG, AGG_IN_, MLP_HID_), lambda h: (h, 0, 0, 0)),
            pl.BlockSpec((1, HPG, MLP_HID_, HD_), lambda h: (h, 0, 0, 0)),
            pl.BlockSpec((H_ * HD_, D_), lambda h: (0, 0)),
            pl.BlockSpec(memory_space=pltpu.SMEM),
        ],
        out_specs=pl.BlockSpec((S_, D_), lambda h: (0, 0)),
        out_shape=jax.ShapeDtypeStruct((S_, D_), jnp.float32),
        scratch_shapes=[
            pltpu.VMEM((S_, VW), jnp.bfloat16),
            pltpu.VMEM((H_, S_, HD_), jnp.bfloat16),
        ],
    )(x, xb, cos_t, sin_t, wqk, wv3, w1b, w2b, wob, jnp.reshape(eps, (1,)))

    return out.reshape(1, S_, D_)
